# Initial kernel scaffold; baseline (speedup 1.0000x reference)
import jax
import jax.numpy as jnp
from jax.experimental import pallas as pl


def kernel(x, ln_w, Wg, bg, w1a_W, w1a_b, w1b_W, w1b_b, w2_W, w2_b):
    raise NotImplementedError("write your pallas kernel here")


# speedup vs baseline: 1.4678x; 1.4678x over previous
import functools

import jax
import jax.numpy as jnp
import numpy as np
from jax.experimental import pallas as pl
from jax.experimental.pallas import tpu as pltpu

_EPS = float(np.finfo(np.float32).eps)


def _moe_body(x_ref, lnw_ref, wg_ref, bg_ref, w1a_ref, w1ab_ref, w1b_ref,
              w1bb_ref, w2_ref, w2b_ref, out_ref,
              xn_ref, i1_ref, i2_ref, g1_ref, g2_ref, *, n_experts):
    e = pl.program_id(0)
    f = pl.program_id(1)
    S = x_ref.shape[0]
    E = n_experts

    @pl.when((e == 0) & (f == 0))
    def _router():
        x = x_ref[...]
        ms = jnp.mean(x * x, axis=-1, keepdims=True)
        xn = x * jax.lax.rsqrt(ms + _EPS) * lnw_ref[...]
        xn_ref[...] = xn
        logits = jax.lax.dot_general(
            xn, wg_ref[...], (((1,), (1,)), ((), ())),
            preferred_element_type=jnp.float32) + bg_ref[...]
        iota = jax.lax.broadcasted_iota(jnp.int32, logits.shape, 1)
        m1 = jnp.max(logits, axis=-1, keepdims=True)
        i1 = jnp.min(jnp.where(logits == m1, iota, E), axis=-1, keepdims=True)
        l2 = jnp.where(iota == i1, -jnp.inf, logits)
        m2 = jnp.max(l2, axis=-1, keepdims=True)
        i2 = jnp.min(jnp.where(l2 == m2, iota, E), axis=-1, keepdims=True)
        p = jnp.exp(m2 - m1)
        g1 = 1.0 / (1.0 + p)
        i1_ref[...] = i1
        i2_ref[...] = i2
        g1_ref[...] = g1
        g2_ref[...] = p * g1

    xn = xn_ref[...]
    w1a = w1a_ref[0]
    w1b = w1b_ref[0]
    a = jax.lax.dot_general(xn, w1a, (((1,), (1,)), ((), ())),
                            preferred_element_type=jnp.float32) + w1ab_ref[0]
    b = jax.lax.dot_general(xn, w1b, (((1,), (1,)), ((), ())),
                            preferred_element_type=jnp.float32) + w1bb_ref[0]
    h = (a / (1.0 + jnp.exp(-a))) * b
    y = jax.lax.dot_general(h, w2_ref[0], (((1,), (1,)), ((), ())),
                            preferred_element_type=jnp.float32)
    ge = (jnp.where(i1_ref[...] == e, g1_ref[...], 0.0)
          + jnp.where(i2_ref[...] == e, g2_ref[...], 0.0))
    contrib = ge * y
    contrib = contrib + jnp.where(f == 0, ge * w2b_ref[0], 0.0)
    prev = jnp.where((e == 0) & (f == 0), 0.0, out_ref[...])
    out_ref[...] = prev + contrib


def kernel(x, ln_w, Wg, bg, w1a_W, w1a_b, w1b_W, w1b_b, w2_W, w2_b):
    B, S, D = x.shape
    E, F = w1a_W.shape[0], w1a_W.shape[1]
    FB = min(F, 512)
    NF = F // FB
    xf = x.reshape(S, D)

    out = pl.pallas_call(
        functools.partial(_moe_body, n_experts=E),
        grid=(E, NF),
        in_specs=[
            pl.BlockSpec((S, D), lambda e, f: (0, 0)),
            pl.BlockSpec((1, D), lambda e, f: (0, 0)),
            pl.BlockSpec((E, D), lambda e, f: (0, 0)),
            pl.BlockSpec((1, E), lambda e, f: (0, 0)),
            pl.BlockSpec((1, FB, D), lambda e, f: (e, f, 0)),
            pl.BlockSpec((1, 1, FB), lambda e, f: (e * NF + f, 0, 0)),
            pl.BlockSpec((1, FB, D), lambda e, f: (e, f, 0)),
            pl.BlockSpec((1, 1, FB), lambda e, f: (e * NF + f, 0, 0)),
            pl.BlockSpec((1, D, FB), lambda e, f: (e, 0, f)),
            pl.BlockSpec((1, 1, D), lambda e, f: (e, 0, 0)),
        ],
        out_specs=pl.BlockSpec((S, D), lambda e, f: (0, 0)),
        out_shape=jax.ShapeDtypeStruct((S, D), jnp.float32),
        scratch_shapes=[
            pltpu.VMEM((S, D), jnp.float32),
            pltpu.VMEM((S, 1), jnp.int32),
            pltpu.VMEM((S, 1), jnp.int32),
            pltpu.VMEM((S, 1), jnp.float32),
            pltpu.VMEM((S, 1), jnp.float32),
        ],
        compiler_params=pltpu.CompilerParams(
            dimension_semantics=("arbitrary", "arbitrary"),
        ),
    )(xf, ln_w.reshape(1, D), Wg, bg.reshape(1, E), w1a_W,
      w1a_b.reshape(E * NF, 1, FB), w1b_W, w1b_b.reshape(E * NF, 1, FB),
      w2_W, w2_b.reshape(E, 1, D))
    return out.reshape(B, S, D)
